# baseline (speedup 1.0000x reference)
import functools

import jax
import jax.numpy as jnp
from jax import lax
from jax.experimental import pallas as pl
from jax.experimental.pallas import tpu as pltpu
from jax.experimental.pallas import tpu_sc as plsc

B, C, H, W, K = 32, 4, 256, 256, 500
HW = H * W
LANES = 16
KPAD = 512
NKV = KPAD // LANES
NQ = 4
HH = H // NQ
NCHUNK = NQ * C
NBUF = 6
NC, NS = 2, 16
LOSS_WEIGHT = 1.0


def _body(table, idxr, maskr, tgtr, outr,
          idx_v, mask_a, mask_b, mask_me, buf_a, buf_b, buf_c, buf_d,
          buf_e, buf_f, tgt_v, out_v, h_v, w_v, e0_v, mf_v, tot_sm, sem):
    cid = lax.axis_index("c")
    sid = lax.axis_index("s")
    b = sid * NC + cid

    bufs = (buf_a, buf_b, buf_c, buf_d, buf_e, buf_f)

    def chunk_src(i):
        return table.at[b, i // NQ, pl.ds((i % NQ) * HH, HH)]

    cps = [pltpu.async_copy(chunk_src(i), bufs[i], sem)
           for i in range(NBUF - 1)]
    pltpu.sync_copy(idxr.at[b], idx_v)
    pltpu.sync_copy(tgtr.at[b], tgt_v)
    pltpu.sync_copy(maskr.at[sid * 2], mask_a)
    pltpu.sync_copy(maskr.at[sid * 2 + 1], mask_b)
    pltpu.sync_copy(maskr.at[b], mask_me)

    iota = lax.iota(jnp.int32, LANES)

    def msum(j, acc):
        kv = j * LANES + iota
        kvec = jnp.minimum(kv, K - 1)
        valid = kv < K
        ga = plsc.load_gather(mask_a, [kvec])
        gb = plsc.load_gather(mask_b, [kvec])
        zero = jnp.zeros((LANES,), jnp.int32)
        return acc + jnp.where(valid, ga + gb, zero)

    acc = lax.fori_loop(0, NKV, msum, jnp.zeros((LANES,), jnp.int32))
    part = jnp.sum(acc)

    @pl.when(sid == 0)
    def _():
        tot_sm[0] = 0

    plsc.subcore_barrier()
    plsc.fetch_and_add(tot_sm.at[0], part, subcore_id=0)
    plsc.subcore_barrier()
    total = plsc.fetch_and_add(tot_sm.at[0], 0, subcore_id=0)

    inv_v = LOSS_WEIGHT / (
        jnp.broadcast_to(total, (LANES,)).astype(jnp.float32) * C + 1e-4)

    def prep(v, carry):
        sl = pl.ds(v * LANES, LANES)
        kvec = jnp.minimum(v * LANES + iota, K - 1)
        iv = plsc.load_gather(idx_v, [kvec])
        me = plsc.load_gather(mask_me, [kvec])
        h_v[sl] = lax.shift_right_logical(iv, 8)
        w_v[sl] = jnp.bitwise_and(iv, 255)
        e0_v[sl] = kvec * C
        mf_v[sl] = me.astype(jnp.float32) * inv_v
        return carry

    lax.fori_loop(0, NKV, prep, 0)

    for i in range(NCHUNK):
        cur = bufs[i % NBUF]
        cps[i].wait()
        if i + NBUF - 1 < NCHUNK:
            cps.append(pltpu.async_copy(
                chunk_src(i + NBUF - 1), bufs[(i + NBUF - 1) % NBUF], sem))
        cc = i // NQ
        lo = (i % NQ) * HH

        def comp(v, carry):
            sl = pl.ds(v * LANES, LANES)
            hl = h_v[sl] - lo
            sel = jnp.logical_and(hl >= 0, hl < HH)
            g = plsc.load_gather(cur, [jnp.bitwise_and(hl, HH - 1), w_v[sl]])
            ev = e0_v[sl] + cc
            tg = plsc.load_gather(tgt_v, [ev])
            res = jnp.abs(g - tg) * mf_v[sl]
            plsc.store_scatter(out_v, [ev], res, mask=sel)
            return carry

        lax.fori_loop(0, NKV, comp, 0)

    pltpu.sync_copy(out_v, outr.at[b])


@jax.jit
def _gather_l1(table, idx, msk, tgt):
    return pl.kernel(
        _body,
        out_type=jax.ShapeDtypeStruct((B, K * C), jnp.float32),
        mesh=plsc.VectorSubcoreMesh(core_axis_name="c", subcore_axis_name="s"),
        compiler_params=pltpu.CompilerParams(
            needs_layout_passes=False,
            skip_device_barrier=True,
            disable_bounds_checks=True,
            disable_semaphore_checks=True,
        ),
        scratch_types=[
            pltpu.VMEM((K,), jnp.int32),
            pltpu.VMEM((K,), jnp.int32),
            pltpu.VMEM((K,), jnp.int32),
            pltpu.VMEM((K,), jnp.int32),
            pltpu.VMEM((HH, W), jnp.float32),
            pltpu.VMEM((HH, W), jnp.float32),
            pltpu.VMEM((HH, W), jnp.float32),
            pltpu.VMEM((HH, W), jnp.float32),
            pltpu.VMEM((HH, W), jnp.float32),
            pltpu.VMEM((HH, W), jnp.float32),
            pltpu.VMEM((K * C,), jnp.float32),
            pltpu.VMEM((K * C,), jnp.float32),
            pltpu.VMEM((KPAD,), jnp.int32),
            pltpu.VMEM((KPAD,), jnp.int32),
            pltpu.VMEM((KPAD,), jnp.int32),
            pltpu.VMEM((KPAD,), jnp.float32),
            pltpu.SMEM((1,), jnp.int32),
            pltpu.SemaphoreType.DMA,
        ],
    )(table, idx, msk, tgt)


def kernel(output, mask, index, target):
    idx = index.astype(jnp.int32)
    msk = mask.astype(jnp.int32)
    tgt = target.reshape(B, K * C)
    out = _gather_l1(output, idx, msk, tgt)
    return out.reshape(B, K, C)


# speedup vs baseline: 1.0108x; 1.0108x over previous
import functools

import jax
import jax.numpy as jnp
from jax import lax
from jax.experimental import pallas as pl
from jax.experimental.pallas import tpu as pltpu
from jax.experimental.pallas import tpu_sc as plsc

B, C, H, W, K = 32, 4, 256, 256, 500
HW = H * W
LANES = 16
KPAD = 512
NKV = KPAD // LANES
NQ = 4
HH = H // NQ
NCHUNK = NQ * C
NBUF = 4
NC, NS = 2, 16
LOSS_WEIGHT = 1.0


def _body(table, idxr, maskr, tgtr, outr,
          idx_v, mask_a, mask_b, mask_me, bufs, tgt_v, out_v,
          h_v, w_v, e0_v, mf_v, tot_sm, sems):
    cid = lax.axis_index("c")
    sid = lax.axis_index("s")
    b = sid * NC + cid

    def chunk_src(i):
        return table.at[b, lax.shift_right_logical(i, 2),
                        pl.ds(jnp.bitwise_and(i, NQ - 1) * HH, HH)]

    def chunk_slot(i):
        return jnp.bitwise_and(i, NBUF - 1)

    for i in range(NBUF - 1):
        pltpu.async_copy(chunk_src(jnp.int32(i)), bufs.at[i], sems.at[i])
    pltpu.sync_copy(idxr.at[b], idx_v)
    pltpu.sync_copy(tgtr.at[b], tgt_v)
    pltpu.sync_copy(maskr.at[sid * 2], mask_a)
    pltpu.sync_copy(maskr.at[sid * 2 + 1], mask_b)
    pltpu.sync_copy(maskr.at[b], mask_me)

    iota = lax.iota(jnp.int32, LANES)

    def msum(j, acc):
        kv = j * LANES + iota
        kvec = jnp.minimum(kv, K - 1)
        valid = kv < K
        ga = plsc.load_gather(mask_a, [kvec])
        gb = plsc.load_gather(mask_b, [kvec])
        zero = jnp.zeros((LANES,), jnp.int32)
        return acc + jnp.where(valid, ga + gb, zero)

    acc = lax.fori_loop(0, NKV, msum, jnp.zeros((LANES,), jnp.int32))
    part = jnp.sum(acc)

    @pl.when(sid == 0)
    def _():
        tot_sm[0] = 0

    plsc.subcore_barrier()
    plsc.fetch_and_add(tot_sm.at[0], part, subcore_id=0)
    plsc.subcore_barrier()
    total = plsc.fetch_and_add(tot_sm.at[0], 0, subcore_id=0)

    inv_v = LOSS_WEIGHT / (
        jnp.broadcast_to(total, (LANES,)).astype(jnp.float32) * C + 1e-4)

    def prep(v, carry):
        sl = pl.ds(v * LANES, LANES)
        kvec = jnp.minimum(v * LANES + iota, K - 1)
        iv = plsc.load_gather(idx_v, [kvec])
        me = plsc.load_gather(mask_me, [kvec])
        h_v[sl] = lax.shift_right_logical(iv, 8)
        w_v[sl] = jnp.bitwise_and(iv, 255)
        e0_v[sl] = kvec * C
        mf_v[sl] = me.astype(jnp.float32) * inv_v
        return carry

    lax.fori_loop(0, NKV, prep, 0)

    def chunk(g, carry):
        slot = chunk_slot(g)
        pltpu.make_async_copy(chunk_src(g), bufs.at[slot],
                              sems.at[slot]).wait()
        nxt = g + NBUF - 1

        @pl.when(nxt < NCHUNK)
        def _():
            ns = chunk_slot(nxt)
            pltpu.async_copy(chunk_src(nxt), bufs.at[ns], sems.at[ns])

        cc = lax.shift_right_logical(g, 2)
        lo = jnp.bitwise_and(g, NQ - 1) * HH
        slot_b = jnp.broadcast_to(slot, (LANES,))

        def comp(v, carry2):
            sl = pl.ds(v * LANES, LANES)
            hl = h_v[sl] - lo
            sel = jnp.logical_and(hl >= 0, hl < HH)
            g_val = plsc.load_gather(
                bufs, [slot_b, jnp.bitwise_and(hl, HH - 1), w_v[sl]])
            ev = e0_v[sl] + cc
            tg = plsc.load_gather(tgt_v, [ev])
            res = jnp.abs(g_val - tg) * mf_v[sl]
            plsc.store_scatter(out_v, [ev], res, mask=sel)
            return carry2

        lax.fori_loop(0, NKV, comp, 0)
        return carry

    lax.fori_loop(0, NCHUNK, chunk, 0)

    pltpu.sync_copy(out_v, outr.at[b])


@jax.jit
def _gather_l1(table, idx, msk, tgt):
    return pl.kernel(
        _body,
        out_type=jax.ShapeDtypeStruct((B, K * C), jnp.float32),
        mesh=plsc.VectorSubcoreMesh(core_axis_name="c", subcore_axis_name="s"),
        compiler_params=pltpu.CompilerParams(needs_layout_passes=False),
        scratch_types=[
            pltpu.VMEM((K,), jnp.int32),
            pltpu.VMEM((K,), jnp.int32),
            pltpu.VMEM((K,), jnp.int32),
            pltpu.VMEM((K,), jnp.int32),
            pltpu.VMEM((NBUF, HH, W), jnp.float32),
            pltpu.VMEM((K * C,), jnp.float32),
            pltpu.VMEM((K * C,), jnp.float32),
            pltpu.VMEM((KPAD,), jnp.int32),
            pltpu.VMEM((KPAD,), jnp.int32),
            pltpu.VMEM((KPAD,), jnp.int32),
            pltpu.VMEM((KPAD,), jnp.float32),
            pltpu.SMEM((1,), jnp.int32),
            pltpu.SemaphoreType.DMA((NBUF,)),
        ],
    )(table, idx, msk, tgt)


def kernel(output, mask, index, target):
    idx = index.astype(jnp.int32)
    msk = mask.astype(jnp.int32)
    tgt = target.reshape(B, K * C)
    out = _gather_l1(output, idx, msk, tgt)
    return out.reshape(B, K, C)


# speedup vs baseline: 1.0847x; 1.0731x over previous
import functools

import jax
import jax.numpy as jnp
from jax import lax
from jax.experimental import pallas as pl
from jax.experimental.pallas import tpu as pltpu
from jax.experimental.pallas import tpu_sc as plsc

B, C, H, W, K = 32, 4, 256, 256, 500
HW = H * W
LANES = 16
KPAD = 512
NKV = KPAD // LANES
NQ = 4
HH = H // NQ
NCHUNK = NQ * C
NBUF = 4
NC, NS = 2, 16
LOSS_WEIGHT = 1.0


def _body(table, idxr, maskr, tgtr, outr,
          idx_v, mask_a, mask_b, mask_me, bufs, tgt_v, out_v,
          h_v, w_v, k_v, mf_v, tot_sm, sems):
    cid = lax.axis_index("c")
    sid = lax.axis_index("s")
    b = sid * NC + cid

    def chunk_src(i):
        return table.at[b, lax.shift_right_logical(i, 2),
                        pl.ds(jnp.bitwise_and(i, NQ - 1) * HH, HH)]

    def chunk_slot(i):
        return jnp.bitwise_and(i, NBUF - 1)

    for i in range(NBUF - 1):
        pltpu.async_copy(chunk_src(jnp.int32(i)), bufs.at[i], sems.at[i])
    pltpu.sync_copy(idxr.at[b], idx_v)
    pltpu.sync_copy(tgtr.at[b], tgt_v)
    pltpu.sync_copy(maskr.at[sid * 2], mask_a)
    pltpu.sync_copy(maskr.at[sid * 2 + 1], mask_b)
    pltpu.sync_copy(maskr.at[b], mask_me)

    iota = lax.iota(jnp.int32, LANES)

    def msum(j, acc):
        kv = j * LANES + iota
        kvec = jnp.minimum(kv, K - 1)
        valid = kv < K
        ga = plsc.load_gather(mask_a, [kvec])
        gb = plsc.load_gather(mask_b, [kvec])
        zero = jnp.zeros((LANES,), jnp.int32)
        return acc + jnp.where(valid, ga + gb, zero)

    acc = lax.fori_loop(0, NKV, msum, jnp.zeros((LANES,), jnp.int32))
    part = jnp.sum(acc)

    @pl.when(sid == 0)
    def _():
        tot_sm[0] = 0

    plsc.subcore_barrier()
    plsc.fetch_and_add(tot_sm.at[0], part, subcore_id=0)
    plsc.subcore_barrier()
    total = plsc.fetch_and_add(tot_sm.at[0], 0, subcore_id=0)

    inv_v = LOSS_WEIGHT / (
        jnp.broadcast_to(total, (LANES,)).astype(jnp.float32) * C + 1e-4)

    def prep(v, carry):
        sl = pl.ds(v * LANES, LANES)
        kvec = jnp.minimum(v * LANES + iota, K - 1)
        iv = plsc.load_gather(idx_v, [kvec])
        me = plsc.load_gather(mask_me, [kvec])
        h_v[sl] = lax.shift_right_logical(iv, 8)
        w_v[sl] = jnp.bitwise_and(iv, 255)
        k_v[sl] = kvec
        mf_v[sl] = me.astype(jnp.float32) * inv_v
        return carry

    lax.fori_loop(0, NKV, prep, 0)

    def chunk(g, carry):
        slot = chunk_slot(g)
        pltpu.make_async_copy(chunk_src(g), bufs.at[slot],
                              sems.at[slot]).wait()
        nxt = g + NBUF - 1

        @pl.when(nxt < NCHUNK)
        def _():
            ns = chunk_slot(nxt)
            pltpu.async_copy(chunk_src(nxt), bufs.at[ns], sems.at[ns])

        cc = lax.shift_right_logical(g, 2)
        lo = jnp.bitwise_and(g, NQ - 1) * HH
        slot_b = jnp.broadcast_to(slot, (LANES,))
        cc_b = jnp.broadcast_to(cc, (LANES,))

        def comp(v, carry2):
            sl = pl.ds(v * LANES, LANES)
            hl = h_v[sl] - lo
            sel = jnp.logical_and(hl >= 0, hl < HH)
            g_val = plsc.load_gather(
                bufs, [slot_b, jnp.bitwise_and(hl, HH - 1), w_v[sl]])
            kv = k_v[sl]
            tg = plsc.load_gather(tgt_v, [cc_b, kv])
            res = jnp.abs(g_val - tg) * mf_v[sl]
            plsc.store_scatter(out_v, [cc_b, kv], res, mask=sel)
            return carry2

        lax.fori_loop(0, NKV, comp, 0)
        return carry

    lax.fori_loop(0, NCHUNK, chunk, 0)

    pltpu.sync_copy(out_v, outr.at[b])


@jax.jit
def _gather_l1(table, idx, msk, tgt):
    return pl.kernel(
        _body,
        out_type=jax.ShapeDtypeStruct((B, C, K), jnp.float32),
        mesh=plsc.VectorSubcoreMesh(core_axis_name="c", subcore_axis_name="s"),
        compiler_params=pltpu.CompilerParams(needs_layout_passes=False),
        scratch_types=[
            pltpu.VMEM((K,), jnp.int32),
            pltpu.VMEM((K,), jnp.int32),
            pltpu.VMEM((K,), jnp.int32),
            pltpu.VMEM((K,), jnp.int32),
            pltpu.VMEM((NBUF, HH, W), jnp.float32),
            pltpu.VMEM((C, K), jnp.float32),
            pltpu.VMEM((C, K), jnp.float32),
            pltpu.VMEM((KPAD,), jnp.int32),
            pltpu.VMEM((KPAD,), jnp.int32),
            pltpu.VMEM((KPAD,), jnp.int32),
            pltpu.VMEM((KPAD,), jnp.float32),
            pltpu.SMEM((1,), jnp.int32),
            pltpu.SemaphoreType.DMA((NBUF,)),
        ],
    )(table, idx, msk, tgt)


def kernel(output, mask, index, target):
    idx = index.astype(jnp.int32)
    msk = mask.astype(jnp.int32)
    tgt = jnp.transpose(target, (0, 2, 1))
    out = _gather_l1(output, idx, msk, tgt)
    return jnp.transpose(out, (0, 2, 1))
